# baseline (speedup 1.0000x reference)
import functools

import jax
import jax.numpy as jnp
from jax import lax
from jax.experimental import pallas as pl
from jax.experimental.pallas import tpu as pltpu
from jax.experimental.pallas import tpu_sc as plsc

VOCAB = 1000000
DIM = 64
B = 4096
L = 200

N = B * L
NC, NS = 2, 16
NW = NC * NS
G = 128
K = 8
CHUNK = K * G
PER_W = N // NW
ROWS_X = N // G
ROWS_PER_W = PER_W // G
N_CHUNKS = PER_W // CHUNK

_mesh = plsc.VectorSubcoreMesh(core_axis_name="c", subcore_axis_name="s")


@functools.partial(
    pl.kernel,
    mesh=_mesh,
    compiler_params=pltpu.CompilerParams(use_tc_tiling_on_sc=False),
    out_type=jax.ShapeDtypeStruct((N, DIM), jnp.float32),
    scratch_types=[
        pltpu.VMEM((K, G), jnp.int32),
        pltpu.VMEM((CHUNK, DIM), jnp.float32),
        pltpu.SemaphoreType.DMA,
    ],
)
def _emb_lookup(x_hbm, table_hbm, out_hbm, idx_v, rows_v, sem):
    wid = lax.axis_index("s") * NC + lax.axis_index("c")
    row_base = wid * ROWS_PER_W

    def body(i, carry):
        row_off = row_base + i * K
        pltpu.sync_copy(x_hbm.at[pl.ds(row_off, K)], idx_v)
        copies = []
        for j in range(K):
            copies.append(
                pltpu.async_copy(
                    table_hbm.at[idx_v.at[j]],
                    rows_v.at[pl.ds(j * G, G)],
                    sem,
                )
            )
        for c in copies:
            c.wait()
        pltpu.sync_copy(rows_v, out_hbm.at[pl.ds(row_off * G, CHUNK)])
        return carry

    lax.fori_loop(0, N_CHUNKS, body, 0)


def kernel(x, table):
    out = _emb_lookup(x.reshape(ROWS_X, G), table)
    return out.reshape(B, L, DIM)


# speedup vs baseline: 1.0166x; 1.0166x over previous
import functools

import jax
import jax.numpy as jnp
from jax import lax
from jax.experimental import pallas as pl
from jax.experimental.pallas import tpu as pltpu
from jax.experimental.pallas import tpu_sc as plsc

VOCAB = 1000000
DIM = 64
B = 4096
L = 200

NC, NS = 2, 16
NW = NC * NS
ROWS_PER_W = B // NW
G0 = 128
G1 = L - G0

_mesh = plsc.VectorSubcoreMesh(core_axis_name="c", subcore_axis_name="s")


@functools.partial(
    pl.kernel,
    mesh=_mesh,
    compiler_params=pltpu.CompilerParams(use_tc_tiling_on_sc=False),
    out_type=jax.ShapeDtypeStruct((B, L, DIM), jnp.float32),
    scratch_types=[
        pltpu.VMEM((ROWS_PER_W, L), jnp.int32),
        pltpu.VMEM((1, L, DIM), jnp.float32),
        pltpu.VMEM((1, L, DIM), jnp.float32),
        pltpu.SemaphoreType.DMA,
        pltpu.SemaphoreType.DMA,
        pltpu.SemaphoreType.DMA,
        pltpu.SemaphoreType.DMA,
    ],
)
def _emb_lookup(x_hbm, table_hbm, out_hbm, idx_v, rows_a, rows_b, sem_ga,
                sem_gb, sem_oa, sem_ob):
    wid = lax.axis_index("s") * NC + lax.axis_index("c")
    row_base = wid * ROWS_PER_W
    pltpu.sync_copy(x_hbm.at[pl.ds(row_base, ROWS_PER_W)], idx_v)

    bufs = ((rows_a, sem_ga, sem_oa), (rows_b, sem_gb, sem_ob))

    def gather_row(r, buf, gsem):
        c0 = pltpu.async_copy(
            table_hbm.at[idx_v.at[r, pl.ds(0, G0)]],
            buf.at[0, pl.ds(0, G0)],
            gsem,
        )
        c1 = pltpu.async_copy(
            table_hbm.at[idx_v.at[r, pl.ds(G0, G1)]],
            buf.at[0, pl.ds(G0, G1)],
            gsem,
        )
        return c0, c1

    def write_row(r, buf, osem):
        return pltpu.async_copy(buf, out_hbm.at[pl.ds(row_base + r, 1)], osem)

    g = gather_row(0, rows_a, sem_ga)

    def body(i, carry):
        for k in range(2):
            r = 2 * i + k
            buf, gsem, osem = bufs[k]
            nbuf, ngsem, _ = bufs[1 - k]
            @pl.when(r + 1 < ROWS_PER_W)
            def _():
                @pl.when(r >= 1)
                def _():
                    pltpu.make_async_copy(
                        nbuf, out_hbm.at[pl.ds(0, 1)], bufs[1 - k][2]
                    ).wait()
                gather_row(r + 1, nbuf, ngsem)
            pltpu.make_async_copy(
                table_hbm.at[idx_v.at[r, pl.ds(0, G0)]],
                buf.at[0, pl.ds(0, G0)],
                gsem,
            ).wait()
            pltpu.make_async_copy(
                table_hbm.at[idx_v.at[r, pl.ds(G0, G1)]],
                buf.at[0, pl.ds(G0, G1)],
                gsem,
            ).wait()
            write_row(r, buf, osem)
        return carry

    lax.fori_loop(0, ROWS_PER_W // 2, body, 0)
    pltpu.make_async_copy(rows_a, out_hbm.at[pl.ds(0, 1)], sem_oa).wait()
    pltpu.make_async_copy(rows_b, out_hbm.at[pl.ds(0, 1)], sem_ob).wait()


def kernel(x, table):
    return _emb_lookup(x, table)


# speedup vs baseline: 1.2430x; 1.2227x over previous
import functools

import jax
import jax.numpy as jnp
from jax import lax
from jax.experimental import pallas as pl
from jax.experimental.pallas import tpu as pltpu
from jax.experimental.pallas import tpu_sc as plsc

VOCAB = 1000000
DIM = 64
B = 4096
L = 200
WIDE = 128
XW = 256

NC, NS = 2, 16
NW = NC * NS
ROWS_PER_W = B // NW
G0 = 128
G1 = L - G0

_mesh = plsc.VectorSubcoreMesh(core_axis_name="c", subcore_axis_name="s")


@functools.partial(
    pl.kernel,
    mesh=_mesh,
    out_type=jax.ShapeDtypeStruct((B * L, WIDE), jnp.float32),
    scratch_types=[
        pltpu.VMEM((ROWS_PER_W, XW), jnp.int32),
        pltpu.VMEM((L, WIDE), jnp.float32),
        pltpu.VMEM((L, WIDE), jnp.float32),
        pltpu.SemaphoreType.DMA,
        pltpu.SemaphoreType.DMA,
        pltpu.SemaphoreType.DMA,
        pltpu.SemaphoreType.DMA,
    ],
)
def _gather(x_hbm, wide_hbm, out_hbm, idx_v, rows_a, rows_b, sem_ga, sem_gb,
            sem_oa, sem_ob):
    wid = lax.axis_index("s") * NC + lax.axis_index("c")
    row_base = wid * ROWS_PER_W
    pltpu.sync_copy(x_hbm.at[pl.ds(row_base, ROWS_PER_W)], idx_v)

    bufs = ((rows_a, sem_ga, sem_oa), (rows_b, sem_gb, sem_ob))

    def gather_row(r, buf, gsem):
        pltpu.async_copy(
            wide_hbm.at[idx_v.at[r, pl.ds(0, G0)]],
            buf.at[pl.ds(0, G0)],
            gsem,
        )
        pltpu.async_copy(
            wide_hbm.at[idx_v.at[r, pl.ds(G0, G1)]],
            buf.at[pl.ds(G0, G1)],
            gsem,
        )

    def wait_gather_row(r, buf, gsem):
        pltpu.make_async_copy(
            wide_hbm.at[idx_v.at[r, pl.ds(0, G0)]],
            buf.at[pl.ds(0, G0)],
            gsem,
        ).wait()
        pltpu.make_async_copy(
            wide_hbm.at[idx_v.at[r, pl.ds(G0, G1)]],
            buf.at[pl.ds(G0, G1)],
            gsem,
        ).wait()

    def write_row(r, buf, osem):
        pltpu.async_copy(
            buf, out_hbm.at[pl.ds((row_base + r) * L, L)], osem
        )

    def wait_write(buf, osem):
        pltpu.make_async_copy(
            buf, out_hbm.at[pl.ds(0, L)], osem
        ).wait()

    gather_row(0, rows_a, sem_ga)

    def body(i, carry):
        for k in range(2):
            r = 2 * i + k
            buf, gsem, osem = bufs[k]
            nbuf, ngsem, nosem = bufs[1 - k]
            @pl.when(r + 1 < ROWS_PER_W)
            def _():
                @pl.when(r >= 1)
                def _():
                    wait_write(nbuf, nosem)
                gather_row(r + 1, nbuf, ngsem)
            wait_gather_row(r, buf, gsem)
            write_row(r, buf, osem)
        return carry

    lax.fori_loop(0, ROWS_PER_W // 2, body, 0)
    wait_write(rows_a, sem_oa)
    wait_write(rows_b, sem_ob)


def kernel(x, table):
    wide = jnp.pad(table, ((0, 0), (0, WIDE - DIM)))
    xp = jnp.pad(x, ((0, 0), (0, XW - L)))
    out = _gather(xp, wide)
    return out[:, :DIM].reshape(B, L, DIM)
